# baseline (speedup 1.0000x reference)
import functools

import jax
import jax.numpy as jnp
from jax import lax
from jax.experimental import pallas as pl
from jax.experimental.pallas import tpu as pltpu
from jax.experimental.pallas import tpu_sc as plsc

_B, _M, _S, _D = 4, 16, 2048, 1024
_CS = 16
_CPR = _S // _CS
_NCHUNK = _M * _CPR
_NW = 32
_G = 2
_KEEP = _M - _B
_GA = _KEEP * _CPR // (_NW * _G)
_GB = _B * _CPR // (_NW * _G)

_mesh = plsc.VectorSubcoreMesh(core_axis_name="c", subcore_axis_name="s")


@functools.partial(
    pl.kernel,
    mesh=_mesh,
    out_type=jax.ShapeDtypeStruct((_NCHUNK, _CS, _D), jnp.float32),
    scratch_types=[
        pltpu.VMEM((_GA, _G), jnp.int32),
        pltpu.VMEM((_GB, _G), jnp.int32),
        pltpu.VMEM((_GB, _G), jnp.int32),
        pltpu.VMEM((_G, _CS, _D), jnp.float32),
        pltpu.VMEM((_G, _CS, _D), jnp.float32),
        pltpu.SemaphoreType.DMA,
        pltpu.SemaphoreType.DMA,
        pltpu.SemaphoreType.DMA,
        pltpu.SemaphoreType.DMA,
    ],
)
def _sc_copy(inp_hbm, inps_hbm, idxa_hbm, idxbs_hbm, idxbd_hbm, out_hbm,
             idxa_v, idxbs_v, idxbd_v, buf0, buf1, g0, g1, s0, s1):
    wid = lax.axis_index("s") * 2 + lax.axis_index("c")
    pltpu.sync_copy(idxa_hbm.at[wid], idxa_v)
    pltpu.sync_copy(idxbs_hbm.at[wid], idxbs_v)
    pltpu.sync_copy(idxbd_hbm.at[wid], idxbd_v)
    bufs = (buf0, buf1)
    gsems = (g0, g1)
    ssems = (s0, s1)

    def run_phase(n, gather_src, scatter_dst):
        hg = [None, None]
        hs = [None, None]
        hg[0] = pltpu.async_copy(gather_src(0), bufs[0], gsems[0])
        for j in range(n):
            sl = j % 2
            nsl = (j + 1) % 2
            if j >= 1:
                hs[nsl].wait()
            if j + 1 < n:
                hg[nsl] = pltpu.async_copy(gather_src(j + 1), bufs[nsl], gsems[nsl])
            hg[sl].wait()
            hs[sl] = pltpu.async_copy(bufs[sl], scatter_dst(j), ssems[sl])
        hs[(n - 1) % 2].wait()

    run_phase(
        _GA,
        lambda j: inps_hbm.at[idxa_v.at[j]],
        lambda j: out_hbm.at[idxa_v.at[j]],
    )
    run_phase(
        _GB,
        lambda j: inp_hbm.at[idxbs_v.at[j]],
        lambda j: out_hbm.at[idxbd_v.at[j]],
    )


def kernel(inp, inps, start_idx):
    s = jnp.asarray(start_idx, jnp.int32)
    rows = jnp.arange(_M, dtype=jnp.int32)
    in_win = jnp.logical_and(rows >= s, rows < s + _B)
    keep = jnp.argsort(jnp.where(in_win, _M + rows, rows))[:_KEEP]
    col = jnp.arange(_CPR, dtype=jnp.int32)[None, :]
    idxa = (keep[:, None] * _CPR + col).reshape(_NW, _GA, _G).astype(jnp.int32)
    win_rows = s + jnp.arange(_B, dtype=jnp.int32)
    idxbd = (win_rows[:, None] * _CPR + col).reshape(_NW, _GB, _G).astype(jnp.int32)
    idxbs = jnp.arange(_B * _CPR, dtype=jnp.int32).reshape(_NW, _GB, _G)
    inp_v = inp.reshape(_B * _CPR, _CS, _D)
    inps_v = inps.reshape(_NCHUNK, _CS, _D)
    out = _sc_copy(inp_v, inps_v, idxa, idxbs, idxbd)
    return out.reshape(_M, _S, _D)


# speedup vs baseline: 1.0035x; 1.0035x over previous
import functools

import jax
import jax.numpy as jnp
from jax import lax
from jax.experimental import pallas as pl
from jax.experimental.pallas import tpu as pltpu
from jax.experimental.pallas import tpu_sc as plsc

_B, _M, _S, _D = 4, 16, 2048, 1024
_CS = 16
_CPR = _S // _CS
_NCHUNK = _M * _CPR
_NW = 32
_G = 2
_KEEP = _M - _B
_GA = _KEEP * _CPR // (_NW * _G)
_GB = _B * _CPR // (_NW * _G)

_mesh = plsc.VectorSubcoreMesh(core_axis_name="c", subcore_axis_name="s")


@functools.partial(
    pl.kernel,
    mesh=_mesh,
    out_type=jax.ShapeDtypeStruct((_NCHUNK, _CS, _D), jnp.float32),
    scratch_types=[
        pltpu.VMEM((_GA, _G), jnp.int32),
        pltpu.VMEM((_GB, _G), jnp.int32),
        pltpu.VMEM((_GB, _G), jnp.int32),
        pltpu.VMEM((_G, _CS, _D), jnp.float32),
        pltpu.VMEM((_G, _CS, _D), jnp.float32),
        pltpu.VMEM((_G, _CS, _D), jnp.float32),
        pltpu.SemaphoreType.DMA,
        pltpu.SemaphoreType.DMA,
        pltpu.SemaphoreType.DMA,
        pltpu.SemaphoreType.DMA,
        pltpu.SemaphoreType.DMA,
        pltpu.SemaphoreType.DMA,
    ],
)
def _sc_copy(inp_hbm, inps_hbm, idxa_hbm, idxbs_hbm, idxbd_hbm, out_hbm,
             idxa_v, idxbs_v, idxbd_v, buf0, buf1, buf2,
             g0, g1, g2, s0, s1, s2):
    wid = lax.axis_index("s") * 2 + lax.axis_index("c")
    pltpu.sync_copy(idxa_hbm.at[wid], idxa_v)
    pltpu.sync_copy(idxbs_hbm.at[wid], idxbs_v)
    pltpu.sync_copy(idxbd_hbm.at[wid], idxbd_v)
    bufs = (buf0, buf1, buf2)
    gsems = (g0, g1, g2)
    ssems = (s0, s1, s2)
    ns = 3
    depth = ns - 1

    def run_phase(n, gather_src, scatter_dst):
        hg = [None] * ns
        hs = [None] * ns
        for j in range(min(depth, n)):
            hg[j % ns] = pltpu.async_copy(gather_src(j), bufs[j % ns], gsems[j % ns])
        for j in range(n):
            sl = j % ns
            if j >= 1:
                hs[(j - 1) % ns].wait()
            if j + depth < n:
                psl = (j + depth) % ns
                hg[psl] = pltpu.async_copy(gather_src(j + depth), bufs[psl], gsems[psl])
            hg[sl].wait()
            hs[sl] = pltpu.async_copy(bufs[sl], scatter_dst(j), ssems[sl])
        hs[(n - 1) % ns].wait()

    run_phase(
        _GA,
        lambda j: inps_hbm.at[idxa_v.at[j]],
        lambda j: out_hbm.at[idxa_v.at[j]],
    )
    run_phase(
        _GB,
        lambda j: inp_hbm.at[idxbs_v.at[j]],
        lambda j: out_hbm.at[idxbd_v.at[j]],
    )


def kernel(inp, inps, start_idx):
    s = jnp.asarray(start_idx, jnp.int32)
    rows = jnp.arange(_M, dtype=jnp.int32)
    in_win = jnp.logical_and(rows >= s, rows < s + _B)
    keep = jnp.argsort(jnp.where(in_win, _M + rows, rows))[:_KEEP]
    col = jnp.arange(_CPR, dtype=jnp.int32)[None, :]
    idxa = (keep[:, None] * _CPR + col).reshape(_NW, _GA, _G).astype(jnp.int32)
    win_rows = s + jnp.arange(_B, dtype=jnp.int32)
    idxbd = (win_rows[:, None] * _CPR + col).reshape(_NW, _GB, _G).astype(jnp.int32)
    idxbs = jnp.arange(_B * _CPR, dtype=jnp.int32).reshape(_NW, _GB, _G)
    inp_v = inp.reshape(_B * _CPR, _CS, _D)
    inps_v = inps.reshape(_NCHUNK, _CS, _D)
    out = _sc_copy(inp_v, inps_v, idxa, idxbs, idxbd)
    return out.reshape(_M, _S, _D)


# speedup vs baseline: 1.3733x; 1.3686x over previous
import jax
import jax.numpy as jnp
from jax.experimental import pallas as pl
from jax.experimental.pallas import tpu as pltpu

_B, _M, _S, _D = 4, 16, 2048, 1024
_S_BLK = 2048


def _body(s_ref, inp_ref, inps_ref, out_ref):
    m = pl.program_id(1)
    s = s_ref[0]
    in_range = jnp.logical_and(m >= s, m < s + _B)

    @pl.when(in_range)
    def _():
        out_ref[...] = inp_ref[...]

    @pl.when(jnp.logical_not(in_range))
    def _():
        out_ref[...] = inps_ref[...]


def _inp_map(c, m, s_ref):
    s = s_ref[0]
    return jnp.clip(m - s, 0, _B - 1), c, 0


def _inps_map(c, m, s_ref):
    s = s_ref[0]
    in_range = jnp.logical_and(m >= s, m < s + _B)
    dead_row = jnp.where(s > 0, s - 1, jnp.minimum(s + _B, _M - 1))
    return jnp.where(in_range, dead_row, m), c, 0


def _out_map(c, m, s_ref):
    return m, c, 0


def kernel(inp, inps, start_idx):
    s = jnp.asarray(start_idx, jnp.int32).reshape((1,))
    grid = (_S // _S_BLK, _M)
    blk = (1, _S_BLK, _D)
    return pl.pallas_call(
        _body,
        grid_spec=pltpu.PrefetchScalarGridSpec(
            num_scalar_prefetch=1,
            grid=grid,
            in_specs=[
                pl.BlockSpec(blk, _inp_map),
                pl.BlockSpec(blk, _inps_map),
            ],
            out_specs=pl.BlockSpec(blk, _out_map),
        ),
        out_shape=jax.ShapeDtypeStruct(inps.shape, inps.dtype),
        compiler_params=pltpu.CompilerParams(vmem_limit_bytes=56 * 1024 * 1024),
    )(s, inp, inps)
